# baseline (speedup 1.0000x reference)
import functools

import jax
import jax.numpy as jnp
from jax import lax
from jax.experimental import pallas as pl
from jax.experimental.pallas import tpu as pltpu
from jax.experimental.pallas import tpu_sc as plsc

LN_EPS = 1e-5

_info = plsc.get_sparse_core_info()
_NC, _NS = _info.num_cores, _info.num_subcores
_NW = _NC * _NS


def _sc_gather(ids_flat, table, chunk):
    n = ids_flat.shape[0]
    d = table.shape[1]
    per_w = n // _NW
    n_chunks = per_w // chunk
    mesh = plsc.VectorSubcoreMesh(core_axis_name="c", subcore_axis_name="s")

    @functools.partial(
        pl.kernel,
        mesh=mesh,
        out_type=jax.ShapeDtypeStruct((n, d), jnp.float32),
        scratch_types=[
            pltpu.VMEM((2, chunk), jnp.int32),
            pltpu.VMEM((2, chunk, d), jnp.float32),
            pltpu.SemaphoreType.DMA,
            pltpu.SemaphoreType.DMA,
        ],
    )
    def k(ids_hbm, table_hbm, out_hbm, idx_v, rows_v, gsem, osem):
        wid = lax.axis_index("s") * _NC + lax.axis_index("c")
        base = wid * per_w

        def issue(slot, ci):
            off = base + ci * chunk
            pltpu.sync_copy(ids_hbm.at[pl.ds(off, chunk)], idx_v.at[slot])
            return pltpu.async_copy(table_hbm.at[idx_v.at[slot]],
                                    rows_v.at[slot], gsem)

        g = issue(0, 0)
        for ci in range(n_chunks):
            slot = ci % 2
            g.wait()
            if ci + 1 < n_chunks:
                g = issue(1 - slot, ci + 1)
            off = base + ci * chunk
            pltpu.async_copy(rows_v.at[slot],
                             out_hbm.at[pl.ds(off, chunk)], osem).wait()

    return k(ids_flat, table)


def _ln_body(g_ref, pos_ref, segf_ref, segtab_ref, gam_ref, bet_ref, o_ref):
    x = g_ref[...] + pos_ref[...]
    s0 = segtab_ref[0:1, :]
    s1 = segtab_ref[1:2, :]
    x = x + s0 + segf_ref[...] * (s1 - s0)
    mean = jnp.mean(x, axis=1, keepdims=True)
    xc = x - mean
    var = jnp.mean(xc * xc, axis=1, keepdims=True)
    y = xc * lax.rsqrt(var + LN_EPS)
    o_ref[...] = y * gam_ref[...] + bet_ref[...]


def _tc_ln(gathered, pos_table, seg_f, segment_table, gamma2d, beta2d,
           block_rows):
    n, d = gathered.shape
    seq = pos_table.shape[0]
    pos_blocks = seq // block_rows
    batch = n // seq

    return pl.pallas_call(
        _ln_body,
        grid=(pos_blocks, batch),
        in_specs=[
            pl.BlockSpec((block_rows, d), lambda p, b: (b * pos_blocks + p, 0)),
            pl.BlockSpec((block_rows, d), lambda p, b: (p, 0)),
            pl.BlockSpec((block_rows, 1), lambda p, b: (b * pos_blocks + p, 0)),
            pl.BlockSpec(segment_table.shape, lambda p, b: (0, 0)),
            pl.BlockSpec((1, d), lambda p, b: (0, 0)),
            pl.BlockSpec((1, d), lambda p, b: (0, 0)),
        ],
        out_specs=pl.BlockSpec((block_rows, d),
                               lambda p, b: (b * pos_blocks + p, 0)),
        out_shape=jax.ShapeDtypeStruct((n, d), jnp.float32),
    )(gathered, pos_table, seg_f, segment_table, gamma2d, beta2d)


def kernel(input_ids, segment_ids, token_table, segment_table, pos_table,
           ln_gamma, ln_beta):
    batch, seq = input_ids.shape
    d = token_table.shape[1]

    ids_flat = input_ids.reshape(-1).astype(jnp.int32)
    seg_f = segment_ids.reshape(-1, 1).astype(jnp.float32)
    gamma2d = ln_gamma.reshape(1, d)
    beta2d = ln_beta.reshape(1, d)

    half = (batch // 2) * seq
    outs = []
    for s in range(2):
        ids_s = lax.slice_in_dim(ids_flat, s * half, (s + 1) * half)
        seg_s = lax.slice_in_dim(seg_f, s * half, (s + 1) * half)
        gathered = _sc_gather(ids_s, token_table, chunk=64)
        outs.append(_tc_ln(gathered, pos_table, seg_s, segment_table,
                           gamma2d, beta2d, block_rows=512))
    return jnp.concatenate(outs, axis=0).reshape(batch, seq, d)


# speedup vs baseline: 1.3465x; 1.3465x over previous
import functools

import jax
import jax.numpy as jnp
from jax import lax
from jax.experimental import pallas as pl
from jax.experimental.pallas import tpu as pltpu
from jax.experimental.pallas import tpu_sc as plsc

LN_EPS = 1e-5

_info = plsc.get_sparse_core_info()
_NC, _NS = _info.num_cores, _info.num_subcores
_NW = _NC * _NS


def _sc_gather(ids_flat, table, chunk):
    n = ids_flat.shape[0]
    d = table.shape[1]
    per_w = n // _NW
    n_chunks = per_w // chunk
    mesh = plsc.VectorSubcoreMesh(core_axis_name="c", subcore_axis_name="s")

    @functools.partial(
        pl.kernel,
        mesh=mesh,
        out_type=jax.ShapeDtypeStruct((n, d), jnp.float32),
        scratch_types=[
            pltpu.VMEM((per_w,), jnp.int32),
            pltpu.VMEM((2, chunk, d), jnp.float32),
            pltpu.SemaphoreType.DMA,
            pltpu.SemaphoreType.DMA,
        ],
    )
    def k(ids_hbm, table_hbm, out_hbm, idx_v, rows_v, gsem, osem):
        wid = lax.axis_index("s") * _NC + lax.axis_index("c")
        base = wid * per_w
        pltpu.sync_copy(ids_hbm.at[pl.ds(base, per_w)], idx_v)

        def issue(slot, ci):
            return pltpu.async_copy(
                table_hbm.at[idx_v.at[pl.ds(ci * chunk, chunk)]],
                rows_v.at[slot], gsem)

        g = issue(0, 0)
        for ci in range(n_chunks):
            slot = ci % 2
            g.wait()
            if ci + 1 < n_chunks:
                g = issue(1 - slot, ci + 1)
            off = base + ci * chunk
            pltpu.async_copy(rows_v.at[slot],
                             out_hbm.at[pl.ds(off, chunk)], osem).wait()

    return k(ids_flat, table)


def _ln_body(g_ref, pos_ref, segf_ref, segtab_ref, gam_ref, bet_ref, o_ref):
    x = g_ref[...] + pos_ref[...]
    s0 = segtab_ref[0:1, :]
    s1 = segtab_ref[1:2, :]
    x = x + s0 + segf_ref[...] * (s1 - s0)
    mean = jnp.mean(x, axis=1, keepdims=True)
    xc = x - mean
    var = jnp.mean(xc * xc, axis=1, keepdims=True)
    y = xc * lax.rsqrt(var + LN_EPS)
    o_ref[...] = y * gam_ref[...] + bet_ref[...]


def _tc_ln(gathered, pos_table, seg_f, segment_table, gamma2d, beta2d,
           block_rows):
    n, d = gathered.shape
    seq = pos_table.shape[0]
    pos_blocks = seq // block_rows
    batch = n // seq

    return pl.pallas_call(
        _ln_body,
        grid=(pos_blocks, batch),
        in_specs=[
            pl.BlockSpec((block_rows, d), lambda p, b: (b * pos_blocks + p, 0)),
            pl.BlockSpec((block_rows, d), lambda p, b: (p, 0)),
            pl.BlockSpec((block_rows, 1), lambda p, b: (b * pos_blocks + p, 0)),
            pl.BlockSpec(segment_table.shape, lambda p, b: (0, 0)),
            pl.BlockSpec((1, d), lambda p, b: (0, 0)),
            pl.BlockSpec((1, d), lambda p, b: (0, 0)),
        ],
        out_specs=pl.BlockSpec((block_rows, d),
                               lambda p, b: (b * pos_blocks + p, 0)),
        out_shape=jax.ShapeDtypeStruct((n, d), jnp.float32),
    )(gathered, pos_table, seg_f, segment_table, gamma2d, beta2d)


def kernel(input_ids, segment_ids, token_table, segment_table, pos_table,
           ln_gamma, ln_beta):
    batch, seq = input_ids.shape
    d = token_table.shape[1]

    ids_flat = input_ids.reshape(-1).astype(jnp.int32)
    seg_f = segment_ids.reshape(-1, 1).astype(jnp.float32)
    gamma2d = ln_gamma.reshape(1, d)
    beta2d = ln_beta.reshape(1, d)

    gathered = _sc_gather(ids_flat, token_table, chunk=64)
    out = _tc_ln(gathered, pos_table, seg_f, segment_table,
                 gamma2d, beta2d, block_rows=1024)
    return out.reshape(batch, seq, d)


# speedup vs baseline: 1.3553x; 1.0065x over previous
import functools

import jax
import jax.numpy as jnp
from jax import lax
from jax.experimental import pallas as pl
from jax.experimental.pallas import tpu as pltpu
from jax.experimental.pallas import tpu_sc as plsc

LN_EPS = 1e-5

_info = plsc.get_sparse_core_info()
_NC, _NS = _info.num_cores, _info.num_subcores
_NW = _NC * _NS


def _sc_gather(ids_flat, table, chunk):
    n = ids_flat.shape[0]
    d = table.shape[1]
    per_w = n // _NW
    n_chunks = per_w // chunk
    mesh = plsc.VectorSubcoreMesh(core_axis_name="c", subcore_axis_name="s")

    @functools.partial(
        pl.kernel,
        mesh=mesh,
        out_type=jax.ShapeDtypeStruct((n, d), jnp.float32),
        scratch_types=[
            pltpu.VMEM((per_w,), jnp.int32),
            pltpu.VMEM((2, chunk, d), jnp.float32),
            pltpu.SemaphoreType.DMA,
            pltpu.SemaphoreType.DMA,
        ],
    )
    def k(ids_hbm, table_hbm, out_hbm, idx_v, rows_v, gsem, osem):
        wid = lax.axis_index("s") * _NC + lax.axis_index("c")
        base = wid * per_w
        pltpu.sync_copy(ids_hbm.at[pl.ds(base, per_w)], idx_v)

        def issue(slot, ci):
            return pltpu.async_copy(
                table_hbm.at[idx_v.at[pl.ds(ci * chunk, chunk)]],
                rows_v.at[slot], gsem)

        g = issue(0, 0)
        for ci in range(n_chunks):
            slot = ci % 2
            g.wait()
            if ci + 1 < n_chunks:
                g = issue(1 - slot, ci + 1)
            off = base + ci * chunk
            pltpu.async_copy(rows_v.at[slot],
                             out_hbm.at[pl.ds(off, chunk)], osem).wait()

    return k(ids_flat, table)


def _ln_body(g_ref, pos_ref, segf_ref, segtab_ref, gam_ref, bet_ref, o_ref):
    x = g_ref[...] + pos_ref[...]
    s0 = segtab_ref[0:1, :]
    s1 = segtab_ref[1:2, :]
    x = x + s0 + segf_ref[...] * (s1 - s0)
    mean = jnp.mean(x, axis=1, keepdims=True)
    xc = x - mean
    var = jnp.mean(xc * xc, axis=1, keepdims=True)
    y = xc * lax.rsqrt(var + LN_EPS)
    o_ref[...] = y * gam_ref[...] + bet_ref[...]


def _tc_ln(gathered, pos_table, seg_f, segment_table, gamma2d, beta2d,
           block_rows):
    n, d = gathered.shape
    seq = pos_table.shape[0]
    pos_blocks = seq // block_rows
    batch = n // seq

    return pl.pallas_call(
        _ln_body,
        grid=(pos_blocks, batch),
        in_specs=[
            pl.BlockSpec((block_rows, d), lambda p, b: (b * pos_blocks + p, 0)),
            pl.BlockSpec((block_rows, d), lambda p, b: (p, 0)),
            pl.BlockSpec((block_rows, 1), lambda p, b: (b * pos_blocks + p, 0)),
            pl.BlockSpec(segment_table.shape, lambda p, b: (0, 0)),
            pl.BlockSpec((1, d), lambda p, b: (0, 0)),
            pl.BlockSpec((1, d), lambda p, b: (0, 0)),
        ],
        out_specs=pl.BlockSpec((block_rows, d),
                               lambda p, b: (b * pos_blocks + p, 0)),
        out_shape=jax.ShapeDtypeStruct((n, d), jnp.float32),
    )(gathered, pos_table, seg_f, segment_table, gamma2d, beta2d)


def kernel(input_ids, segment_ids, token_table, segment_table, pos_table,
           ln_gamma, ln_beta):
    batch, seq = input_ids.shape
    d = token_table.shape[1]

    ids_flat = input_ids.reshape(-1).astype(jnp.int32)
    seg_f = segment_ids.reshape(-1, 1).astype(jnp.float32)
    gamma2d = ln_gamma.reshape(1, d)
    beta2d = ln_beta.reshape(1, d)

    gathered = _sc_gather(ids_flat, token_table, chunk=64)
    out = _tc_ln(gathered, pos_table, seg_f, segment_table,
                 gamma2d, beta2d, block_rows=2048)
    return out.reshape(batch, seq, d)
